# Initial kernel scaffold; baseline (speedup 1.0000x reference)
import functools

import jax
import jax.numpy as jnp
from jax import lax
from jax.experimental import pallas as pl
from jax.experimental.pallas import tpu as pltpu
from jax.experimental.pallas import tpu_sc as plsc

N = 10000
E = 320000
D = 128
H = 128
G = 16

NC = 2
NS = 16
NW = NC * NS

C = 128
NCH = 79
EPT = NCH * C
EPAD = NW * EPT
NP = 10240
RPT = NP // NS
CW = 16
PADROW = N


def _sc_mesh():
    return plsc.VectorSubcoreMesh(core_axis_name="c", subcore_axis_name="s")


def _sc_degree(dst3):
    @functools.partial(
        pl.kernel,
        out_type=jax.ShapeDtypeStruct((NC, NP, CW), jnp.float32),
        mesh=_sc_mesh(),
        scratch_types=[
            pltpu.VMEM((NCH, C), jnp.int32),
            pltpu.VMEM((C, CW), jnp.float32),
            pltpu.VMEM((C, CW), jnp.float32),
            pltpu.VMEM_SHARED((NP, CW), jnp.float32),
        ],
    )
    def k(dst_hbm, out_hbm, idx_v, ones_v, stg_v, acc):
        c = lax.axis_index("c")
        s = lax.axis_index("s")
        g = c * NS + s

        def fill(i, carry):
            ones_v[i] = jnp.ones((CW,), jnp.float32)
            stg_v[i] = jnp.zeros((CW,), jnp.float32)
            return carry

        lax.fori_loop(0, C, fill, 0)
        for kk in range(RPT // C):
            pltpu.sync_copy(stg_v, acc.at[pl.ds(s * RPT + kk * C, C)])
        pltpu.sync_copy(dst_hbm.at[g], idx_v)
        plsc.subcore_barrier()

        def body(j, carry):
            pltpu.sync_copy(ones_v, acc.at[idx_v.at[j]], add=True)
            return carry

        lax.fori_loop(0, NCH, body, 0)
        plsc.subcore_barrier()
        for kk in range(RPT // C):
            pltpu.sync_copy(acc.at[pl.ds(s * RPT + kk * C, C)], stg_v)
            pltpu.sync_copy(stg_v, out_hbm.at[c].at[pl.ds(s * RPT + kk * C, C)])

    return k(dst3)


def _sc_scatter(y, src3, dst3):
    @functools.partial(
        pl.kernel,
        out_type=jax.ShapeDtypeStruct((NC, NP, H), jnp.float32),
        mesh=_sc_mesh(),
        scratch_types=[
            pltpu.VMEM((NCH, C), jnp.int32),
            pltpu.VMEM((NCH, C), jnp.int32),
            pltpu.VMEM((C, H), jnp.float32),
            pltpu.VMEM_SHARED((NP, H), jnp.float32),
            pltpu.SemaphoreType.DMA,
        ],
    )
    def k(y_hbm, src_hbm, dst_hbm, out_hbm, si, di, rbuf, acc, gsem):
        c = lax.axis_index("c")
        s = lax.axis_index("s")
        g = c * NS + s

        def zf(i, carry):
            for kk in range(H // 16):
                rbuf[i, pl.ds(kk * 16, 16)] = jnp.zeros((16,), jnp.float32)
            return carry

        lax.fori_loop(0, C, zf, 0)
        for kk in range(RPT // C):
            pltpu.sync_copy(rbuf, acc.at[pl.ds(s * RPT + kk * C, C)])
        pltpu.sync_copy(src_hbm.at[g], si)
        pltpu.sync_copy(dst_hbm.at[g], di)
        plsc.subcore_barrier()

        def body(j, carry):
            pltpu.async_copy(y_hbm.at[si.at[j]], rbuf, gsem).wait()
            pltpu.sync_copy(rbuf, acc.at[di.at[j]], add=True)
            return carry

        lax.fori_loop(0, NCH, body, 0)
        plsc.subcore_barrier()
        for kk in range(RPT // C):
            pltpu.sync_copy(acc.at[pl.ds(s * RPT + kk * C, C)], rbuf)
            pltpu.sync_copy(rbuf, out_hbm.at[c].at[pl.ds(s * RPT + kk * C, C)])

    return k(y, src3, dst3)


R = 2000
NBLK = N // R


def _dinv_block(cnt_ref):
    deg = cnt_ref[0][:, 0:1] + cnt_ref[1][:, 0:1] + 1.0
    return lax.rsqrt(deg)


def _stage0_body(x_ref, cnt_ref, we_ref, be_ref, w0_ref, y_ref):
    dinv = _dinv_block(cnt_ref)
    h = jnp.maximum(
        jnp.dot(x_ref[...], we_ref[...], preferred_element_type=jnp.float32)
        + be_ref[...],
        0.0,
    )
    y_ref[...] = (
        jnp.dot(h, w0_ref[...], preferred_element_type=jnp.float32) * dinv
    )


def _tc_stage0(x, cnt, W_emb, b_emb, W0):
    return pl.pallas_call(
        _stage0_body,
        grid=(NBLK,),
        in_specs=[
            pl.BlockSpec((R, D), lambda i: (i, 0)),
            pl.BlockSpec((NC, R, CW), lambda i: (0, i, 0)),
            pl.BlockSpec((D, H), lambda i: (0, 0)),
            pl.BlockSpec((1, H), lambda i: (0, 0)),
            pl.BlockSpec((H, H), lambda i: (0, 0)),
        ],
        out_specs=pl.BlockSpec((R, H), lambda i: (i, 0)),
        out_shape=jax.ShapeDtypeStruct((N, H), jnp.float32),
    )(x, cnt, W_emb, b_emb, W0)


def _stage_body(p_ref, yp_ref, cnt_ref, b_ref, w_ref, y_ref):
    dinv = _dinv_block(cnt_ref)
    conv = (p_ref[0] + p_ref[1] + yp_ref[...]) * dinv + b_ref[...]
    h = jnp.maximum(conv, 0.0)
    y_ref[...] = (
        jnp.dot(h, w_ref[...], preferred_element_type=jnp.float32) * dinv
    )


def _tc_stage(p, y_prev, cnt, b, W):
    return pl.pallas_call(
        _stage_body,
        grid=(NBLK,),
        in_specs=[
            pl.BlockSpec((NC, R, H), lambda i: (0, i, 0)),
            pl.BlockSpec((R, H), lambda i: (i, 0)),
            pl.BlockSpec((NC, R, CW), lambda i: (0, i, 0)),
            pl.BlockSpec((1, H), lambda i: (0, 0)),
            pl.BlockSpec((H, H), lambda i: (0, 0)),
        ],
        out_specs=pl.BlockSpec((R, H), lambda i: (i, 0)),
        out_shape=jax.ShapeDtypeStruct((N, H), jnp.float32),
    )(p, y_prev, cnt, b, W)


def _final_body(p_ref, yp_ref, cnt_ref, b_ref, batch_ref, wv_ref, bv_ref,
                out_ref, gsum, gcnt):
    i = pl.program_id(0)

    @pl.when(i == 0)
    def _init():
        gsum[...] = jnp.zeros((G, H), jnp.float32)
        gcnt[...] = jnp.zeros((G, 128), jnp.float32)
        out_ref[...] = jnp.zeros((G, 1), jnp.float32)

    dinv = _dinv_block(cnt_ref)
    conv = (p_ref[0] + p_ref[1] + yp_ref[...]) * dinv + b_ref[...]
    h = jnp.maximum(conv, 0.0)
    gids = lax.broadcasted_iota(jnp.int32, (1, G), 1)
    mask = (batch_ref[...] == gids).astype(jnp.float32)
    gsum[...] += lax.dot_general(
        mask, h, (((0,), (0,)), ((), ())),
        preferred_element_type=jnp.float32,
    )
    gcnt[...] += lax.dot_general(
        mask, jnp.ones((R, 128), jnp.float32), (((0,), (0,)), ((), ())),
        preferred_element_type=jnp.float32,
    )

    @pl.when(i == NBLK - 1)
    def _fin():
        emb = gsum[...] / jnp.maximum(gcnt[...], 1.0)
        out_ref[...] = (
            jnp.dot(emb, wv_ref[...], preferred_element_type=jnp.float32)
            + bv_ref[...]
        )


def _tc_final(p, y_prev, cnt, b, batch2, W_val, b_val):
    return pl.pallas_call(
        _final_body,
        grid=(NBLK,),
        in_specs=[
            pl.BlockSpec((NC, R, H), lambda i: (0, i, 0)),
            pl.BlockSpec((R, H), lambda i: (i, 0)),
            pl.BlockSpec((NC, R, CW), lambda i: (0, i, 0)),
            pl.BlockSpec((1, H), lambda i: (0, 0)),
            pl.BlockSpec((R, 1), lambda i: (i, 0)),
            pl.BlockSpec((H, 1), lambda i: (0, 0)),
            pl.BlockSpec((1, 1), lambda i: (0, 0)),
        ],
        out_specs=pl.BlockSpec((G, 1), lambda i: (0, 0)),
        out_shape=jax.ShapeDtypeStruct((G, 1), jnp.float32),
        scratch_shapes=[
            pltpu.VMEM((G, H), jnp.float32),
            pltpu.VMEM((G, 128), jnp.float32),
        ],
    )(p, y_prev, cnt, b, batch2, W_val, b_val)


def kernel(x, edge_index, batch, W_emb, b_emb, W0, b0, W1, b1, W2, b2,
           W_val, b_val):
    src = edge_index[0]
    dst = edge_index[1]
    pad = EPAD - E
    srcp = jnp.concatenate([src, jnp.zeros((pad,), jnp.int32)])
    dstp = jnp.concatenate([dst, jnp.full((pad,), PADROW, jnp.int32)])
    src3 = srcp.reshape(NW, NCH, C)
    dst3 = dstp.reshape(NW, NCH, C)

    b_emb2 = b_emb.reshape(1, H)
    b02 = b0.reshape(1, H)
    b12 = b1.reshape(1, H)
    b22 = b2.reshape(1, H)
    bv2 = b_val.reshape(1, 1)
    batch2 = batch.reshape(N, 1)

    cnt = _sc_degree(dst3)
    y0 = _tc_stage0(x, cnt, W_emb, b_emb2, W0)
    p = _sc_scatter(y0, src3, dst3)
    y1 = _tc_stage(p, y0, cnt, b02, W1)
    p = _sc_scatter(y1, src3, dst3)
    y2 = _tc_stage(p, y1, cnt, b12, W2)
    p = _sc_scatter(y2, src3, dst3)
    value = _tc_final(p, y2, cnt, b22, batch2, W_val, b_val)
    return value


# speedup vs baseline: 9.0904x; 9.0904x over previous
import functools

import jax
import jax.numpy as jnp
from jax import lax
from jax.experimental import pallas as pl
from jax.experimental.pallas import tpu as pltpu
from jax.experimental.pallas import tpu_sc as plsc

N = 10000
E = 320000
D = 128
H = 128
G = 16

NC = 2
NS = 16
NW = NC * NS

C = 128
NCH = 79
EPT = NCH * C
EPAD = NW * EPT
NP = 10240
RPT = NP // NS
CW = 128
PADROW = N


def _sc_mesh():
    return plsc.VectorSubcoreMesh(
        core_axis_name="c", subcore_axis_name="s",
        num_cores=NC, num_subcores=NS,
    )


def _sc_scatter(y, src3, dst3):
    @functools.partial(
        pl.kernel,
        out_type=jax.ShapeDtypeStruct((NC, NP, H), jnp.float32),
        mesh=_sc_mesh(),
        scratch_types=[
            pltpu.VMEM((NCH, C), jnp.int32),
            pltpu.VMEM((NCH, C), jnp.int32),
            pltpu.VMEM((C, H), jnp.float32),
            pltpu.VMEM_SHARED((NP, H), jnp.float32),
            pltpu.SemaphoreType.DMA,
        ],
    )
    def k(y_hbm, src_hbm, dst_hbm, out_hbm, si, di, rbuf, acc, gsem):
        c = lax.axis_index("c")
        s = lax.axis_index("s")
        g = c * NS + s

        def zf(i, carry):
            for kk in range(H // 16):
                rbuf[i, pl.ds(kk * 16, 16)] = jnp.zeros((16,), jnp.float32)
            return carry

        lax.fori_loop(0, C, zf, 0)
        for kk in range(RPT // C):
            pltpu.sync_copy(rbuf, acc.at[pl.ds(s * RPT + kk * C, C)])
        pltpu.sync_copy(src_hbm.at[g], si)
        pltpu.sync_copy(dst_hbm.at[g], di)
        plsc.subcore_barrier()

        def body(j, carry):
            pltpu.async_copy(y_hbm.at[si.at[j]], rbuf, gsem).wait()
            pltpu.sync_copy(rbuf, acc.at[di.at[j]], add=True)
            return carry

        lax.fori_loop(0, NCH, body, 0)
        plsc.subcore_barrier()
        for kk in range(RPT // C):
            pltpu.sync_copy(acc.at[pl.ds(s * RPT + kk * C, C)], rbuf)
            pltpu.sync_copy(rbuf, out_hbm.at[c].at[pl.ds(s * RPT + kk * C, C)])

    return k(y, src3, dst3)


R = 2000
NBLK = N // R


def _dinv_block(cnt_ref):
    deg = cnt_ref[0][:, 0:1] + cnt_ref[1][:, 0:1] + 1.0
    return lax.rsqrt(deg)


def _stage0_body(x_ref, cnt_ref, we_ref, be_ref, w0_ref, y_ref):
    dinv = _dinv_block(cnt_ref)
    h = jnp.maximum(
        jnp.dot(x_ref[...], we_ref[...], preferred_element_type=jnp.float32)
        + be_ref[...],
        0.0,
    )
    y_ref[...] = (
        jnp.dot(h, w0_ref[...], preferred_element_type=jnp.float32) * dinv
    )


def _tc_stage0(x, cnt, W_emb, b_emb, W0):
    return pl.pallas_call(
        _stage0_body,
        grid=(NBLK,),
        in_specs=[
            pl.BlockSpec((R, D), lambda i: (i, 0)),
            pl.BlockSpec((NC, R, CW), lambda i: (0, i, 0)),
            pl.BlockSpec((D, H), lambda i: (0, 0)),
            pl.BlockSpec((1, H), lambda i: (0, 0)),
            pl.BlockSpec((H, H), lambda i: (0, 0)),
        ],
        out_specs=pl.BlockSpec((R, H), lambda i: (i, 0)),
        out_shape=jax.ShapeDtypeStruct((N, H), jnp.float32),
    )(x, cnt, W_emb, b_emb, W0)


def _stage_body(p_ref, yp_ref, cnt_ref, b_ref, w_ref, y_ref):
    dinv = _dinv_block(cnt_ref)
    conv = (p_ref[0] + p_ref[1] + yp_ref[...]) * dinv + b_ref[...]
    h = jnp.maximum(conv, 0.0)
    y_ref[...] = (
        jnp.dot(h, w_ref[...], preferred_element_type=jnp.float32) * dinv
    )


def _tc_stage(p, y_prev, cnt, b, W):
    return pl.pallas_call(
        _stage_body,
        grid=(NBLK,),
        in_specs=[
            pl.BlockSpec((NC, R, H), lambda i: (0, i, 0)),
            pl.BlockSpec((R, H), lambda i: (i, 0)),
            pl.BlockSpec((NC, R, CW), lambda i: (0, i, 0)),
            pl.BlockSpec((1, H), lambda i: (0, 0)),
            pl.BlockSpec((H, H), lambda i: (0, 0)),
        ],
        out_specs=pl.BlockSpec((R, H), lambda i: (i, 0)),
        out_shape=jax.ShapeDtypeStruct((N, H), jnp.float32),
    )(p, y_prev, cnt, b, W)


def _final_body(p_ref, yp_ref, cnt_ref, b_ref, batch_ref, wv_ref, bv_ref,
                out_ref, gsum, gcnt):
    i = pl.program_id(0)

    @pl.when(i == 0)
    def _init():
        gsum[...] = jnp.zeros((G, H), jnp.float32)
        gcnt[...] = jnp.zeros((G, 128), jnp.float32)
        out_ref[...] = jnp.zeros((G, 1), jnp.float32)

    dinv = _dinv_block(cnt_ref)
    conv = (p_ref[0] + p_ref[1] + yp_ref[...]) * dinv + b_ref[...]
    h = jnp.maximum(conv, 0.0)
    gids = lax.broadcasted_iota(jnp.int32, (1, G), 1)
    mask = (batch_ref[...] == gids).astype(jnp.float32)
    gsum[...] += lax.dot_general(
        mask, h, (((0,), (0,)), ((), ())),
        preferred_element_type=jnp.float32,
    )
    gcnt[...] += lax.dot_general(
        mask, jnp.ones((R, 128), jnp.float32), (((0,), (0,)), ((), ())),
        preferred_element_type=jnp.float32,
    )

    @pl.when(i == NBLK - 1)
    def _fin():
        emb = gsum[...] / jnp.maximum(gcnt[...], 1.0)
        out_ref[...] = (
            jnp.dot(emb, wv_ref[...], preferred_element_type=jnp.float32)
            + bv_ref[...]
        )


def _tc_final(p, y_prev, cnt, b, batch2, W_val, b_val):
    return pl.pallas_call(
        _final_body,
        grid=(NBLK,),
        in_specs=[
            pl.BlockSpec((NC, R, H), lambda i: (0, i, 0)),
            pl.BlockSpec((R, H), lambda i: (i, 0)),
            pl.BlockSpec((NC, R, CW), lambda i: (0, i, 0)),
            pl.BlockSpec((1, H), lambda i: (0, 0)),
            pl.BlockSpec((R, 1), lambda i: (i, 0)),
            pl.BlockSpec((H, 1), lambda i: (0, 0)),
            pl.BlockSpec((1, 1), lambda i: (0, 0)),
        ],
        out_specs=pl.BlockSpec((G, 1), lambda i: (0, 0)),
        out_shape=jax.ShapeDtypeStruct((G, 1), jnp.float32),
        scratch_shapes=[
            pltpu.VMEM((G, H), jnp.float32),
            pltpu.VMEM((G, 128), jnp.float32),
        ],
    )(p, y_prev, cnt, b, batch2, W_val, b_val)


def kernel(x, edge_index, batch, W_emb, b_emb, W0, b0, W1, b1, W2, b2,
           W_val, b_val):
    src = edge_index[0]
    dst = edge_index[1]
    pad = EPAD - E
    srcp = jnp.concatenate([src, jnp.zeros((pad,), jnp.int32)])
    dstp = jnp.concatenate([dst, jnp.full((pad,), PADROW, jnp.int32)])
    src3 = srcp.reshape(NW, NCH, C)
    dst3 = dstp.reshape(NW, NCH, C)

    b_emb2 = b_emb.reshape(1, H)
    b02 = b0.reshape(1, H)
    b12 = b1.reshape(1, H)
    b22 = b2.reshape(1, H)
    bv2 = b_val.reshape(1, 1)
    batch2 = batch.reshape(N, 1)

    onesN = jnp.ones((N, H), jnp.float32)
    cnt = _sc_scatter(onesN, src3, dst3)
    y0 = _tc_stage0(x, cnt, W_emb, b_emb2, W0)
    p = _sc_scatter(y0, src3, dst3)
    y1 = _tc_stage(p, y0, cnt, b02, W1)
    p = _sc_scatter(y1, src3, dst3)
    y2 = _tc_stage(p, y1, cnt, b12, W2)
    p = _sc_scatter(y2, src3, dst3)
    value = _tc_final(p, y2, cnt, b22, batch2, W_val, bv2)
    return value
